# baseline (speedup 1.0000x reference)
import functools

import jax
import jax.numpy as jnp
from jax.experimental import pallas as pl
from jax.experimental.pallas import tpu as pltpu

_LANE = 128


def _fused_kernel(idx_ref, w1_hbm, w2_hbm, out_hbm,
                  hid_ref, hvm_ref, w2vm_ref, obuf_ref,
                  sem_g, sem_w, sem_o,
                  *, m, s_chunks, nc, tc):
    c = pl.program_id(0)
    base = c * m

    for b in range(m):
        pltpu.make_async_copy(
            w1_hbm.at[idx_ref[base + b]], hid_ref.at[b], sem_g).start()

    for n in range(nc):
        cols = pl.ds(n * tc, tc)
        pltpu.make_async_copy(
            w2_hbm.at[:, cols], w2vm_ref.at[:, cols], sem_w.at[n]).start()

    for b in range(m):
        pltpu.make_async_copy(
            w1_hbm.at[idx_ref[base]], hid_ref.at[0], sem_g).wait()
    hvm_ref[...] = jnp.concatenate(
        [hid_ref[:, s, :] for s in range(s_chunks)], axis=1)

    for n in range(nc):
        k = n % 2
        cols = pl.ds(n * tc, tc)
        pltpu.make_async_copy(
            w2_hbm.at[:, cols], w2vm_ref.at[:, cols], sem_w.at[n]).wait()
        if n >= 2:
            pltpu.make_async_copy(
                obuf_ref.at[k], out_hbm.at[pl.ds(base, m), cols],
                sem_o.at[k]).wait()
        obuf_ref[k] = jnp.dot(hvm_ref[...], w2vm_ref[:, cols],
                              preferred_element_type=jnp.float32)
        pltpu.make_async_copy(
            obuf_ref.at[k], out_hbm.at[pl.ds(base, m), cols],
            sem_o.at[k]).start(priority=1)

    for n in range(max(nc - 2, 0), nc):
        k = n % 2
        cols = pl.ds(n * tc, tc)
        pltpu.make_async_copy(
            obuf_ref.at[k], out_hbm.at[pl.ds(base, m), cols],
            sem_o.at[k]).wait()


def kernel(idx, w1, w2):
    (bsz,) = idx.shape
    voc, emb = w1.shape
    assert w2.shape == (emb, voc) and emb % _LANE == 0 and bsz % 2 == 0
    s_chunks = emb // _LANE
    w1_rows = w1.reshape(voc, s_chunks, _LANE)
    m = bsz // 2
    tc = min(2048, voc)
    nc = voc // tc
    assert nc * tc == voc

    grid_spec = pltpu.PrefetchScalarGridSpec(
        num_scalar_prefetch=1,
        grid=(2,),
        in_specs=[
            pl.BlockSpec(memory_space=pl.ANY),
            pl.BlockSpec(memory_space=pl.ANY),
        ],
        out_specs=pl.BlockSpec(memory_space=pl.ANY),
        scratch_shapes=[
            pltpu.VMEM((m, s_chunks, _LANE), jnp.float32),
            pltpu.VMEM((m, emb), jnp.float32),
            pltpu.VMEM((emb, voc), jnp.float32),
            pltpu.VMEM((2, m, tc), jnp.float32),
            pltpu.SemaphoreType.DMA,
            pltpu.SemaphoreType.DMA((nc,)),
            pltpu.SemaphoreType.DMA((2,)),
        ],
    )
    return pl.pallas_call(
        functools.partial(_fused_kernel, m=m, s_chunks=s_chunks, nc=nc, tc=tc),
        grid_spec=grid_spec,
        out_shape=jax.ShapeDtypeStruct((bsz, voc), jnp.float32),
        compiler_params=pltpu.CompilerParams(
            dimension_semantics=("parallel",),
            disable_bounds_checks=True,
        ),
    )(idx, w1_rows, w2)


# speedup vs baseline: 1.8286x; 1.8286x over previous
import functools

import jax
import jax.numpy as jnp
from jax.experimental import pallas as pl
from jax.experimental.pallas import tpu as pltpu


def _fused_kernel(idx_ref, w1_hbm, w2_hbm, out_hbm,
                  hid_ref, w2vm_ref, obuf_ref,
                  sem_g, sem_w, sem_o,
                  *, m, nc, tc):
    c = pl.program_id(0)
    base = c * m

    for b in range(m):
        pltpu.make_async_copy(
            w1_hbm.at[pl.ds(idx_ref[base + b], 1), :],
            hid_ref.at[pl.ds(b, 1), :], sem_g).start()

    for n in range(nc):
        cols = pl.ds(n * tc, tc)
        pltpu.make_async_copy(
            w2_hbm.at[:, cols], w2vm_ref.at[:, cols],
            sem_w.at[n]).start(priority=1)

    for b in range(m):
        pltpu.make_async_copy(
            w1_hbm.at[pl.ds(idx_ref[base], 1), :],
            hid_ref.at[pl.ds(0, 1), :], sem_g).wait()

    for n in range(nc):
        k = n % 2
        cols = pl.ds(n * tc, tc)
        pltpu.make_async_copy(
            w2_hbm.at[:, cols], w2vm_ref.at[:, cols], sem_w.at[n]).wait()
        if n >= 2:
            pltpu.make_async_copy(
                obuf_ref.at[k], out_hbm.at[pl.ds(base, m), cols],
                sem_o.at[k]).wait()
        obuf_ref[k] = jnp.dot(hid_ref[...], w2vm_ref[:, cols],
                              preferred_element_type=jnp.float32)
        pltpu.make_async_copy(
            obuf_ref.at[k], out_hbm.at[pl.ds(base, m), cols],
            sem_o.at[k]).start()

    for n in range(max(nc - 2, 0), nc):
        k = n % 2
        cols = pl.ds(n * tc, tc)
        pltpu.make_async_copy(
            obuf_ref.at[k], out_hbm.at[pl.ds(base, m), cols],
            sem_o.at[k]).wait()


def kernel(idx, w1, w2):
    (bsz,) = idx.shape
    voc, emb = w1.shape
    assert w2.shape == (emb, voc) and bsz % 2 == 0
    m = bsz // 2
    tc = min(2048, voc)
    nc = voc // tc
    assert nc * tc == voc

    grid_spec = pltpu.PrefetchScalarGridSpec(
        num_scalar_prefetch=1,
        grid=(2,),
        in_specs=[
            pl.BlockSpec(memory_space=pl.ANY),
            pl.BlockSpec(memory_space=pl.ANY),
        ],
        out_specs=pl.BlockSpec(memory_space=pl.ANY),
        scratch_shapes=[
            pltpu.VMEM((m, emb), jnp.float32),
            pltpu.VMEM((emb, voc), jnp.float32),
            pltpu.VMEM((2, m, tc), jnp.float32),
            pltpu.SemaphoreType.DMA,
            pltpu.SemaphoreType.DMA((nc,)),
            pltpu.SemaphoreType.DMA((2,)),
        ],
    )
    return pl.pallas_call(
        functools.partial(_fused_kernel, m=m, nc=nc, tc=tc),
        grid_spec=grid_spec,
        out_shape=jax.ShapeDtypeStruct((bsz, voc), jnp.float32),
        compiler_params=pltpu.CompilerParams(
            dimension_semantics=("parallel",),
            disable_bounds_checks=True,
        ),
    )(idx, w1, w2)
